# Initial kernel scaffold; baseline (speedup 1.0000x reference)
import jax
import jax.numpy as jnp
from jax.experimental import pallas as pl


def kernel(X, R, batch_index, W_lin, b_lin, F1, F2, U1, bu1, U2, bu2):
    raise NotImplementedError("write your pallas kernel here")


# speedup vs baseline: 38.9550x; 38.9550x over previous
import functools

import jax
import jax.numpy as jnp
from jax import lax
from jax.experimental import pallas as pl
from jax.experimental.pallas import tpu as pltpu

_TILE = 128
_CH = 8


def _band_kernel(lo_ref, hi_ref, a_ref, b_ref, bc_ref, br_ref, x_ref,
                 cen_ref, f1_ref, f2_ref, u1_ref, bu1_ref, u2_ref, bu2_ref,
                 out_ref, *, r2, inv_delta, hidden):
    t = pl.program_id(0)
    lo = lo_ref[t]
    hi = hi_ref[t]
    ad = a_ref[...]
    bd = bc_ref[...]
    jd = t * _TILE + lax.broadcasted_iota(jnp.int32, (_TILE, 1), 0)
    cen = cen_ref[...]
    f1 = f1_ref[...]
    f2 = f2_ref[...]

    def body(c, acc):
        s0 = pl.multiple_of(c * _TILE, _TILE)
        bs = b_ref[:, pl.ds(s0, _TILE)]
        d2 = jnp.dot(ad, bs, preferred_element_type=jnp.float32,
                     precision=lax.Precision.HIGHEST)
        bsr = br_ref[:, pl.ds(s0, _TILE)]
        is_ = s0 + lax.broadcasted_iota(jnp.int32, (1, _TILE), 1)
        m = (d2 < r2) & (bd == bsr) & (jd != is_)
        mf = m.astype(jnp.float32)
        xs = x_ref[pl.ds(s0, _TILE), :]
        for k in range(_TILE // _CH):
            cols = d2[:, k * _CH:(k + 1) * _CH]
            d2col = jnp.concatenate(
                [cols[:, i:i + 1] for i in range(_CH)], axis=0)
            z = (d2col - cen) * inv_delta
            phi = jnp.exp(-0.5 * z * z)
            h1 = jnp.maximum(
                jnp.dot(phi, f1, preferred_element_type=jnp.float32,
                        precision=lax.Precision.HIGHEST), 0.0)
            mk = jnp.maximum(
                jnp.dot(h1, f2, preferred_element_type=jnp.float32,
                        precision=lax.Precision.HIGHEST), 0.0)
            for i in range(_CH):
                s = k * _CH + i
                acc = acc + (mf[:, s:s + 1] * xs[s:s + 1, :]
                             * mk[i * _TILE:(i + 1) * _TILE, :])
        return acc

    acc = lax.fori_loop(lo, hi, body,
                        jnp.zeros((_TILE, hidden), jnp.float32))
    h = jnp.maximum(
        jnp.dot(acc, u1_ref[...], preferred_element_type=jnp.float32,
                precision=lax.Precision.HIGHEST)
        + bu1_ref[...], 0.0)
    out_ref[...] = (jnp.dot(h, u2_ref[...], preferred_element_type=jnp.float32,
                            precision=lax.Precision.HIGHEST)
                    + bu2_ref[...])


def kernel(X, R, batch_index, W_lin, b_lin, F1, F2, U1, bu1, U2, bu2):
    del W_lin, b_lin
    v, hidden = X.shape
    num_bases = F1.shape[0]
    radius = 0.4
    d_min, d_max = 0.0, 0.16
    r2 = radius * radius
    delta = (d_max - d_min) / (num_bases - 1)

    nt = (v + _TILE - 1) // _TILE
    vp = nt * _TILE
    p = vp - v

    bi = batch_index.astype(jnp.int32)
    pad_b = bi[-1] + 1 + jnp.arange(p, dtype=jnp.int32)
    bip = jnp.concatenate([bi, pad_b])
    rp = jnp.concatenate([R, jnp.zeros((p, 3), R.dtype)])
    xp = jnp.concatenate([X, jnp.zeros((p, hidden), X.dtype)])

    norms = jnp.sum(rp * rp, axis=1, keepdims=True)
    ones = jnp.ones_like(norms)
    zeros3 = jnp.zeros((vp, 3), jnp.float32)
    a_geo = jnp.concatenate([-2.0 * rp, norms, ones, zeros3], axis=1)
    b_geo = jnp.concatenate([rp, ones, norms, zeros3], axis=1).T

    firsts = bip[0::_TILE]
    lasts = bip[_TILE - 1::_TILE]
    lo_node = jnp.searchsorted(bip, firsts, side="left").astype(jnp.int32)
    hi_node = jnp.searchsorted(bip, lasts, side="right").astype(jnp.int32)
    lo_t = lo_node // _TILE
    hi_t = (hi_node + _TILE - 1) // _TILE

    centers = jnp.linspace(d_min, d_max, num_bases,
                           dtype=jnp.float32).reshape(1, num_bases)

    full = lambda shape: pl.BlockSpec(shape, lambda t, lo, hi: (0, 0))
    grid_spec = pltpu.PrefetchScalarGridSpec(
        num_scalar_prefetch=2,
        grid=(nt,),
        in_specs=[
            pl.BlockSpec((_TILE, 8), lambda t, lo, hi: (t, 0)),
            full((8, vp)),
            pl.BlockSpec((_TILE, 1), lambda t, lo, hi: (t, 0)),
            full((1, vp)),
            full((vp, hidden)),
            full((1, num_bases)),
            full((num_bases, hidden)),
            full((hidden, hidden)),
            full((hidden, hidden)),
            full((1, hidden)),
            full((hidden, hidden)),
            full((1, hidden)),
        ],
        out_specs=pl.BlockSpec((_TILE, hidden), lambda t, lo, hi: (t, 0)),
    )

    out = pl.pallas_call(
        functools.partial(_band_kernel, r2=r2, inv_delta=1.0 / delta,
                          hidden=hidden),
        grid_spec=grid_spec,
        out_shape=jax.ShapeDtypeStruct((vp, hidden), jnp.float32),
    )(lo_t, hi_t, a_geo, b_geo, bip[:, None], bip[None, :], xp, centers,
      F1, F2, U1, bu1.reshape(1, hidden), U2, bu2.reshape(1, hidden))
    return out[:v]


# speedup vs baseline: 158.0774x; 4.0579x over previous
import functools

import jax
import jax.numpy as jnp
from jax import lax
from jax.experimental import pallas as pl
from jax.experimental.pallas import tpu as pltpu

_TILE = 128
_CH = 8


def _band_kernel(lo_ref, hi_ref, a_ref, b_ref, bc_ref, br_ref, x_ref,
                 cen_ref, f1_ref, f2_ref, u1_ref, bu1_ref, u2_ref, bu2_ref,
                 out_ref, *, r2, inv_delta, hidden):
    t = pl.program_id(0)
    lo = lo_ref[t]
    hi = hi_ref[t]
    ad = a_ref[...]
    bd = bc_ref[...]
    jd = t * _TILE + lax.broadcasted_iota(jnp.int32, (_TILE, 1), 0)
    cen = cen_ref[...]
    f1 = f1_ref[...]
    f2 = f2_ref[...]

    def body(c, acc):
        s0 = pl.multiple_of(c * _TILE, _TILE)
        bs = b_ref[:, pl.ds(s0, _TILE)]
        d2 = jnp.dot(ad, bs, preferred_element_type=jnp.float32,
                     precision=lax.Precision.HIGHEST)
        bsr = br_ref[:, pl.ds(s0, _TILE)]
        is_ = s0 + lax.broadcasted_iota(jnp.int32, (1, _TILE), 1)
        m = (d2 < r2) & (bd == bsr) & (jd != is_)
        d2m = jnp.where(m, d2, 1e4)
        xs = x_ref[pl.ds(s0, _TILE), :]
        for k in range(_TILE // _CH):
            cols = d2m[:, k * _CH:(k + 1) * _CH]
            d2col = jnp.concatenate(
                [cols[:, i:i + 1] for i in range(_CH)], axis=0)
            z = (d2col - cen) * inv_delta
            phi = jnp.exp(-0.5 * z * z)
            h1 = jnp.maximum(
                jnp.dot(phi, f1, preferred_element_type=jnp.float32), 0.0)
            mk = jnp.maximum(
                jnp.dot(h1, f2, preferred_element_type=jnp.float32), 0.0)
            for i in range(_CH):
                s = k * _CH + i
                acc = acc + xs[s:s + 1, :] * mk[i * _TILE:(i + 1) * _TILE, :]
        return acc

    acc = lax.fori_loop(lo, hi, body,
                        jnp.zeros((_TILE, hidden), jnp.float32))
    h = jnp.maximum(
        jnp.dot(acc, u1_ref[...], preferred_element_type=jnp.float32,
                precision=lax.Precision.HIGHEST)
        + bu1_ref[...], 0.0)
    out_ref[...] = (jnp.dot(h, u2_ref[...], preferred_element_type=jnp.float32,
                            precision=lax.Precision.HIGHEST)
                    + bu2_ref[...])


def kernel(X, R, batch_index, W_lin, b_lin, F1, F2, U1, bu1, U2, bu2):
    del W_lin, b_lin
    v, hidden = X.shape
    num_bases = F1.shape[0]
    radius = 0.4
    d_min, d_max = 0.0, 0.16
    r2 = radius * radius
    delta = (d_max - d_min) / (num_bases - 1)

    nt = (v + _TILE - 1) // _TILE
    vp = nt * _TILE
    p = vp - v

    bi = batch_index.astype(jnp.int32)
    pad_b = bi[-1] + 1 + jnp.arange(p, dtype=jnp.int32)
    bip = jnp.concatenate([bi, pad_b])
    rp = jnp.concatenate([R, jnp.zeros((p, 3), R.dtype)])
    xp = jnp.concatenate([X, jnp.zeros((p, hidden), X.dtype)])

    norms = jnp.sum(rp * rp, axis=1, keepdims=True)
    ones = jnp.ones_like(norms)
    zeros3 = jnp.zeros((vp, 3), jnp.float32)
    a_geo = jnp.concatenate([-2.0 * rp, norms, ones, zeros3], axis=1)
    b_geo = jnp.concatenate([rp, ones, norms, zeros3], axis=1).T

    firsts = bip[0::_TILE]
    lasts = bip[_TILE - 1::_TILE]
    lo_node = jnp.searchsorted(bip, firsts, side="left").astype(jnp.int32)
    hi_node = jnp.searchsorted(bip, lasts, side="right").astype(jnp.int32)
    lo_t = lo_node // _TILE
    hi_t = (hi_node + _TILE - 1) // _TILE

    centers = jnp.linspace(d_min, d_max, num_bases,
                           dtype=jnp.float32).reshape(1, num_bases)

    full = lambda shape: pl.BlockSpec(shape, lambda t, lo, hi: (0, 0))
    grid_spec = pltpu.PrefetchScalarGridSpec(
        num_scalar_prefetch=2,
        grid=(nt,),
        in_specs=[
            pl.BlockSpec((_TILE, 8), lambda t, lo, hi: (t, 0)),
            full((8, vp)),
            pl.BlockSpec((_TILE, 1), lambda t, lo, hi: (t, 0)),
            full((1, vp)),
            full((vp, hidden)),
            full((1, num_bases)),
            full((num_bases, hidden)),
            full((hidden, hidden)),
            full((hidden, hidden)),
            full((1, hidden)),
            full((hidden, hidden)),
            full((1, hidden)),
        ],
        out_specs=pl.BlockSpec((_TILE, hidden), lambda t, lo, hi: (t, 0)),
    )

    out = pl.pallas_call(
        functools.partial(_band_kernel, r2=r2, inv_delta=1.0 / delta,
                          hidden=hidden),
        grid_spec=grid_spec,
        out_shape=jax.ShapeDtypeStruct((vp, hidden), jnp.float32),
    )(lo_t, hi_t, a_geo, b_geo, bip[:, None], bip[None, :], xp, centers,
      F1, F2, U1, bu1.reshape(1, hidden), U2, bu2.reshape(1, hidden))
    return out[:v]
